# baseline (speedup 1.0000x reference)
import functools

import jax
import jax.numpy as jnp
from jax import lax
from jax.experimental import pallas as pl
from jax.experimental.pallas import tpu as pltpu
from jax.experimental.pallas import tpu_sc as plsc

B, S, EMB, ENT, V, L = 32, 128, 128, 128, 100000, 42
ND = EMB + ENT
BS = B * S
CH = 32
SP = 0.05
TOPK = 3
BN_N = float(B * S * ND)

F32 = jnp.float32


def _sc_gather(table, idx):
    info = plsc.get_sparse_core_info()
    nw = info.num_cores * info.num_subcores
    rows_per_w = BS // nw
    mesh = plsc.VectorSubcoreMesh(core_axis_name="c", subcore_axis_name="s")

    @functools.partial(
        pl.kernel,
        mesh=mesh,
        out_type=jax.ShapeDtypeStruct((BS, EMB), F32),
        scratch_types=[
            pltpu.VMEM((rows_per_w,), jnp.int32),
            pltpu.VMEM((rows_per_w, EMB), F32),
            pltpu.SemaphoreType.DMA,
        ],
    )
    def gk(table_hbm, idx_hbm, out_hbm, idx_v, rows_v, sem):
        wid = lax.axis_index("s") * info.num_cores + lax.axis_index("c")
        base = wid * rows_per_w
        pltpu.sync_copy(idx_hbm.at[pl.ds(base, rows_per_w)], idx_v)
        pltpu.async_copy(table_hbm.at[idx_v], rows_v, sem).wait()
        pltpu.sync_copy(rows_v, out_hbm.at[pl.ds(base, rows_per_w)])

    return gk(table, idx)


def _feat_body(seq_ref, m1_ref, m2_ref, wep_ref, bep_ref, e_ref, out_ref):
    seq = seq_ref[...]
    rowb = lax.broadcasted_iota(jnp.int32, (B, BS), 0)
    colb = lax.broadcasted_iota(jnp.int32, (B, BS), 1) // S

    def pool(m):
        denom = jnp.sum(m, axis=1, keepdims=True) + 1e-13
        tiled = jnp.concatenate([m] * B, axis=1)
        sel = jnp.where(colb == rowb, tiled, 0.0)
        p = jax.lax.dot_general(sel, seq, (((1,), (0,)), ((), ())))
        return p / denom

    p1 = pool(m1_ref[...])
    p2 = pool(m2_ref[...])
    ef = jax.lax.dot_general(jnp.concatenate([p1, p2], axis=1), wep_ref[...],
                             (((1,), (0,)), ((), ()))) + bep_ref[...]
    right = jax.lax.dot_general(e_ref[...], ef, (((1,), (0,)), ((), ())))
    out_ref[...] = jnp.concatenate([seq, right], axis=1)


def _feat(seq, m1, m2, wep, bep_r, e_mat):
    return pl.pallas_call(
        _feat_body,
        out_shape=jax.ShapeDtypeStruct((BS, ND), F32),
    )(seq, m1, m2, wep, bep_r, e_mat)


def _col_shifts(x):
    zc = jnp.zeros((x.shape[0], 1), F32)
    xl = jnp.concatenate([zc, x[:, :-1]], axis=1)
    xr = jnp.concatenate([x[:, 1:], zc], axis=1)
    return xl, x, xr


def _conv1(x, a1_ref, cb1_rep):
    xs = _col_shifts(x)
    h = cb1_rep
    for k in range(3):
        h = h + jnp.dot(a1_ref[k], xs[k], preferred_element_type=F32)
    return h


def _stats_body(feat_ref, a1_ref, cb1_ref, e_ref, s_ref, q_ref):
    i = pl.program_id(0)

    @pl.when(i == 0)
    def _():
        s_ref[...] = jnp.zeros_like(s_ref)
        q_ref[...] = jnp.zeros_like(q_ref)

    h = _conv1(feat_ref[...], a1_ref, cb1_ref[...])
    rs = jnp.sum(h, axis=1, keepdims=True)
    rq = jnp.sum(h * h, axis=1, keepdims=True)
    e = e_ref[...]
    s_ref[...] += jax.lax.dot_general(e, rs, (((0,), (0,)), ((), ())))
    q_ref[...] += jax.lax.dot_general(e, rq, (((0,), (0,)), ((), ())))


def _stats(feat, a1, cb1_rep, e_mat):
    return pl.pallas_call(
        _stats_body,
        grid=(B,),
        in_specs=[
            pl.BlockSpec((S, ND), lambda i: (i, 0)),
            pl.BlockSpec((3, BS, S), lambda i: (0, 0, 0)),
            pl.BlockSpec((BS, 1), lambda i: (0, 0)),
            pl.BlockSpec((BS, B), lambda i: (0, 0)),
        ],
        out_specs=[
            pl.BlockSpec((B, 1), lambda i: (0, 0)),
            pl.BlockSpec((B, 1), lambda i: (0, 0)),
        ],
        out_shape=[
            jax.ShapeDtypeStruct((B, 1), F32),
            jax.ShapeDtypeStruct((B, 1), F32),
        ],
    )(feat, a1, cb1_rep, e_mat)


def _block_body(feat_ref, gw_ref, gb_ref, s_ref, q_ref, bng_ref, bnb_ref,
                a1_ref, a2_ref, cb1_ref, cb2_ref, e_ref, wg1_ref, wg2_ref,
                bg_ref, out_ref):
    x = feat_ref[...]

    n = jnp.sqrt(jnp.sum(x * x, axis=1, keepdims=True))
    xn = x / jnp.maximum(n, 1e-12)
    sim = jax.lax.dot_general(xn, xn, (((1,), (1,)), ((), ())))

    lane = lax.broadcasted_iota(jnp.int32, (S, S), 1)
    simw = sim
    amask = jnp.zeros_like(sim)
    for _ in range(TOPK):
        m = jnp.max(simw, axis=1, keepdims=True)
        am = jnp.min(jnp.where(simw == m, lane, S), axis=1, keepdims=True)
        hit = lane == am
        amask = amask + jnp.where(hit, m, 0.0)
        simw = jnp.where(hit, -3.0, simw)

    gw = gw_ref[...]
    dw = ((gw > SP).astype(F32) - (gw < -SP).astype(F32)) * SP
    support = jnp.dot(x, dw, preferred_element_type=F32)
    g1 = jnp.dot(amask, support, preferred_element_type=F32)
    g2 = jax.lax.dot_general(amask, support, (((0,), (0,)), ((), ())))
    gcn = jax.nn.relu(0.5 * (g1 + g2) + gb_ref[...])

    mu = s_ref[...] / BN_N
    var = q_ref[...] / BN_N - mu * mu
    a32 = bng_ref[...] * jax.lax.rsqrt(var + 1e-5)
    d32 = bnb_ref[...] - mu * a32
    e = e_ref[...]
    abn = jnp.dot(e, a32, preferred_element_type=F32)
    dbn = jnp.dot(e, d32, preferred_element_type=F32)

    h = _conv1(x, a1_ref, cb1_ref[...])
    hn = jax.nn.relu(h * abn + dbn)
    hs = _col_shifts(hn)
    co = cb2_ref[0, 0]
    for k in range(3):
        co = co + jnp.dot(a2_ref[k], hs[k], preferred_element_type=F32)

    out = gcn + co

    glin = (jnp.sum(x * wg1_ref[...], axis=1, keepdims=True)
            + jnp.sum(out * wg2_ref[...], axis=1, keepdims=True)
            + bg_ref[0, 0])
    g = 1.0 / (1.0 + jnp.exp(-glin))
    out_ref[...] = g * x + (1.0 - g) * out


def _block(feat, gw, gb_r, s, q, bng_c, bnb_c, a1, a2, cb1_rep, cb2_r,
           e_mat, wg1, wg2, bg_r):
    zero2 = lambda i: (0, 0)
    return pl.pallas_call(
        _block_body,
        grid=(B,),
        in_specs=[
            pl.BlockSpec((S, ND), lambda i: (i, 0)),
            pl.BlockSpec((ND, ND), zero2),
            pl.BlockSpec((1, ND), zero2),
            pl.BlockSpec((B, 1), zero2),
            pl.BlockSpec((B, 1), zero2),
            pl.BlockSpec((B, 1), zero2),
            pl.BlockSpec((B, 1), zero2),
            pl.BlockSpec((3, BS, S), lambda i: (0, 0, 0)),
            pl.BlockSpec((3, S, BS), lambda i: (0, 0, 0)),
            pl.BlockSpec((BS, 1), zero2),
            pl.BlockSpec((1, 1), zero2, memory_space=pltpu.SMEM),
            pl.BlockSpec((BS, B), zero2),
            pl.BlockSpec((1, ND), zero2),
            pl.BlockSpec((1, ND), zero2),
            pl.BlockSpec((1, 1), zero2, memory_space=pltpu.SMEM),
        ],
        out_specs=pl.BlockSpec((S, ND), lambda i: (i, 0)),
        out_shape=jax.ShapeDtypeStruct((BS, ND), F32),
    )(feat, gw, gb_r, s, q, bng_c, bnb_c, a1, a2, cb1_rep, cb2_r,
      e_mat, wg1, wg2, bg_r)


def _head_body(feat_ref, e_ref, w1_ref, b1_ref, w2_ref, b2_ref, out_ref):
    pooled = jax.lax.dot_general(e_ref[...], feat_ref[...],
                                 (((0,), (0,)), ((), ()))) * (1.0 / S)
    r = jax.nn.relu(jnp.dot(pooled, w1_ref[...],
                            preferred_element_type=F32) + b1_ref[...])
    out_ref[...] = jnp.dot(r, w2_ref[...],
                           preferred_element_type=F32) + b2_ref[...]


def _head(feat, e_mat, w1, b1_r, w2, b2_r):
    return pl.pallas_call(
        _head_body,
        out_shape=jax.ShapeDtypeStruct((B, L), F32),
    )(feat, e_mat, w1, b1_r, w2, b2_r)


def _banded(w3x3, transpose):
    r = jnp.arange(BS)
    i = (r % S)[:, None]
    c = (r // S)[:, None]
    ip = jnp.arange(S)[None, :]
    dh = ip - i + 1
    valid = (dh >= 0) & (dh <= 2)
    dhc = jnp.clip(dh, 0, 2)
    idx = (2 - dhc) if transpose else dhc
    mats = []
    for dwk in range(3):
        vals = w3x3[c, idx, dwk] * valid
        mats.append(vals.T if transpose else vals)
    return jnp.stack(mats).astype(F32)


def _dense_forward(seq, e1_mask, e2_mask, W_ep, b_ep, blocks,
                   W_gate, b_gate, Wc1, bc1, Wc2, bc2):
    e_mat = (jnp.arange(BS)[:, None] // S == jnp.arange(B)[None, :]).astype(F32)
    feat = _feat(seq, e1_mask, e2_mask, W_ep, b_ep.reshape(1, ENT), e_mat)

    wg1 = W_gate[:ND, 0].reshape(1, ND)
    wg2 = W_gate[ND:, 0].reshape(1, ND)
    bg_r = b_gate.reshape(1, 1)

    for (gw, gb, cw1, cb1, bng, bnb, cw2, cb2) in blocks:
        a1 = _banded(cw1.reshape(CH, 3, 3), transpose=False)
        a2 = _banded(cw2.reshape(CH, 3, 3), transpose=True)
        cb1_rep = jnp.repeat(cb1, S).reshape(BS, 1).astype(F32)
        s, q = _stats(feat, a1, cb1_rep, e_mat)
        feat = _block(feat, gw, gb.reshape(1, ND), s, q,
                      bng.reshape(B, 1), bnb.reshape(B, 1), a1, a2,
                      cb1_rep, cb2.reshape(1, 1), e_mat, wg1, wg2, bg_r)

    return _head(feat, e_mat, Wc1, bc1.reshape(1, ND), Wc2, bc2.reshape(1, L))


def kernel(input_ids, e1_mask, e2_mask, emb, W_ep, b_ep,
           gcn_w0, gcn_b0, cw1_0, cb1_0, bng0, bnb0, cw2_0, cb2_0,
           gcn_w1, gcn_b1, cw1_1, cb1_1, bng1, bnb1, cw2_1, cb2_1,
           W_gate, b_gate, Wc1, bc1, Wc2, bc2):
    ids = input_ids.reshape(BS).astype(jnp.int32)
    seq = _sc_gather(emb, ids)
    blocks = [(gcn_w0, gcn_b0, cw1_0, cb1_0, bng0, bnb0, cw2_0, cb2_0),
              (gcn_w1, gcn_b1, cw1_1, cb1_1, bng1, bnb1, cw2_1, cb2_1)]
    return _dense_forward(seq, e1_mask, e2_mask, W_ep, b_ep, blocks,
                          W_gate, b_gate, Wc1, bc1, Wc2, bc2)


# speedup vs baseline: 142.0321x; 142.0321x over previous
import functools

import jax
import jax.numpy as jnp
from jax import lax
from jax.experimental import pallas as pl
from jax.experimental.pallas import tpu as pltpu
from jax.experimental.pallas import tpu_sc as plsc

B, S, EMB, ENT, V, L = 32, 128, 128, 128, 100000, 42
ND = EMB + ENT
BS = B * S
CH = 32
SP = 0.05
TOPK = 3
BN_N = float(B * S * ND)

F32 = jnp.float32


def _sc_gather(table, idx):
    info = plsc.get_sparse_core_info()
    nw = info.num_cores * info.num_subcores
    rows_per_w = BS // nw
    mesh = plsc.VectorSubcoreMesh(core_axis_name="c", subcore_axis_name="s")

    @functools.partial(
        pl.kernel,
        mesh=mesh,
        out_type=jax.ShapeDtypeStruct((BS, EMB), F32),
        scratch_types=[
            pltpu.VMEM((rows_per_w,), jnp.int32),
            pltpu.VMEM((rows_per_w, EMB), F32),
            pltpu.SemaphoreType.DMA,
        ],
    )
    def gk(table_hbm, idx_hbm, out_hbm, idx_v, rows_v, sem):
        wid = lax.axis_index("s") * info.num_cores + lax.axis_index("c")
        base = wid * rows_per_w
        pltpu.sync_copy(idx_hbm.at[pl.ds(base, rows_per_w)], idx_v)
        pltpu.async_copy(table_hbm.at[idx_v], rows_v, sem).wait()
        pltpu.sync_copy(rows_v, out_hbm.at[pl.ds(base, rows_per_w)])

    return gk(table, idx)


def _feat_body(seq_ref, m1_ref, m2_ref, wep_ref, bep_ref, e_ref, out_ref):
    seq = seq_ref[...]
    rowb = lax.broadcasted_iota(jnp.int32, (B, BS), 0)
    colb = lax.broadcasted_iota(jnp.int32, (B, BS), 1) // S

    def pool(m):
        denom = jnp.sum(m, axis=1, keepdims=True) + 1e-13
        tiled = jnp.concatenate([m] * B, axis=1)
        sel = jnp.where(colb == rowb, tiled, 0.0)
        p = jax.lax.dot_general(sel, seq, (((1,), (0,)), ((), ())))
        return p / denom

    p1 = pool(m1_ref[...])
    p2 = pool(m2_ref[...])
    ef = jax.lax.dot_general(jnp.concatenate([p1, p2], axis=1), wep_ref[...],
                             (((1,), (0,)), ((), ()))) + bep_ref[...]
    right = jax.lax.dot_general(e_ref[...], ef, (((1,), (0,)), ((), ())))
    out_ref[...] = jnp.concatenate([seq, right], axis=1)


def _feat(seq, m1, m2, wep, bep_r, e_mat):
    return pl.pallas_call(
        _feat_body,
        out_shape=jax.ShapeDtypeStruct((BS, ND), F32),
    )(seq, m1, m2, wep, bep_r, e_mat)


def _col_shifts(x):
    zc = jnp.zeros((x.shape[0], 1), F32)
    xl = jnp.concatenate([zc, x[:, :-1]], axis=1)
    xr = jnp.concatenate([x[:, 1:], zc], axis=1)
    return xl, x, xr


def _conv1(x, a1_ref, cb1_rep):
    xs = _col_shifts(x)
    h = cb1_rep
    for k in range(3):
        h = h + jnp.dot(a1_ref[k], xs[k], preferred_element_type=F32)
    return h


def _stats_body(feat_ref, a1_ref, cb1_ref, e_ref, s_ref, q_ref):
    i = pl.program_id(0)

    @pl.when(i == 0)
    def _():
        s_ref[...] = jnp.zeros_like(s_ref)
        q_ref[...] = jnp.zeros_like(q_ref)

    h = _conv1(feat_ref[...], a1_ref, cb1_ref[...])
    rs = jnp.sum(h, axis=1, keepdims=True)
    rq = jnp.sum(h * h, axis=1, keepdims=True)
    e = e_ref[...]
    s_ref[...] += jax.lax.dot_general(e, rs, (((0,), (0,)), ((), ())))
    q_ref[...] += jax.lax.dot_general(e, rq, (((0,), (0,)), ((), ())))


def _stats(feat, a1, cb1_rep, e_mat):
    return pl.pallas_call(
        _stats_body,
        grid=(B,),
        in_specs=[
            pl.BlockSpec((S, ND), lambda i: (i, 0)),
            pl.BlockSpec((3, BS, S), lambda i: (0, 0, 0)),
            pl.BlockSpec((BS, 1), lambda i: (0, 0)),
            pl.BlockSpec((BS, B), lambda i: (0, 0)),
        ],
        out_specs=[
            pl.BlockSpec((B, 1), lambda i: (0, 0)),
            pl.BlockSpec((B, 1), lambda i: (0, 0)),
        ],
        out_shape=[
            jax.ShapeDtypeStruct((B, 1), F32),
            jax.ShapeDtypeStruct((B, 1), F32),
        ],
    )(feat, a1, cb1_rep, e_mat)


def _block_body(feat_ref, gw_ref, gb_ref, s_ref, q_ref, bng_ref, bnb_ref,
                a1_ref, a2_ref, cb1_ref, cb2_ref, e_ref, wg1_ref, wg2_ref,
                bg_ref, out_ref):
    x = feat_ref[...]

    n = jnp.sqrt(jnp.sum(x * x, axis=1, keepdims=True))
    xn = x / jnp.maximum(n, 1e-12)
    sim = jax.lax.dot_general(xn, xn, (((1,), (1,)), ((), ())))

    lane = lax.broadcasted_iota(jnp.int32, (S, S), 1)
    simw = sim
    amask = jnp.zeros_like(sim)
    for _ in range(TOPK):
        m = jnp.max(simw, axis=1, keepdims=True)
        am = jnp.min(jnp.where(simw == m, lane, S), axis=1, keepdims=True)
        hit = lane == am
        amask = amask + jnp.where(hit, m, 0.0)
        simw = jnp.where(hit, -3.0, simw)

    gw = gw_ref[...]
    dw = ((gw > SP).astype(F32) - (gw < -SP).astype(F32)) * SP
    support = jnp.dot(x, dw, preferred_element_type=F32)
    g1 = jnp.dot(amask, support, preferred_element_type=F32)
    g2 = jax.lax.dot_general(amask, support, (((0,), (0,)), ((), ())))
    gcn = jax.nn.relu(0.5 * (g1 + g2) + gb_ref[...])

    mu = s_ref[...] / BN_N
    var = q_ref[...] / BN_N - mu * mu
    a32 = bng_ref[...] * jax.lax.rsqrt(var + 1e-5)
    d32 = bnb_ref[...] - mu * a32
    e = e_ref[...]
    abn = jnp.dot(e, a32, preferred_element_type=F32)
    dbn = jnp.dot(e, d32, preferred_element_type=F32)

    h = _conv1(x, a1_ref, cb1_ref[...])
    hn = jax.nn.relu(h * abn + dbn)
    hs = _col_shifts(hn)
    co = cb2_ref[0, 0]
    for k in range(3):
        co = co + jnp.dot(a2_ref[k], hs[k], preferred_element_type=F32)

    out = gcn + co

    glin = (jnp.sum(x * wg1_ref[...], axis=1, keepdims=True)
            + jnp.sum(out * wg2_ref[...], axis=1, keepdims=True)
            + bg_ref[0, 0])
    g = 1.0 / (1.0 + jnp.exp(-glin))
    out_ref[...] = g * x + (1.0 - g) * out


def _block(feat, gw, gb_r, s, q, bng_c, bnb_c, a1, a2, cb1_rep, cb2_r,
           e_mat, wg1, wg2, bg_r):
    zero2 = lambda i: (0, 0)
    return pl.pallas_call(
        _block_body,
        grid=(B,),
        in_specs=[
            pl.BlockSpec((S, ND), lambda i: (i, 0)),
            pl.BlockSpec((ND, ND), zero2),
            pl.BlockSpec((1, ND), zero2),
            pl.BlockSpec((B, 1), zero2),
            pl.BlockSpec((B, 1), zero2),
            pl.BlockSpec((B, 1), zero2),
            pl.BlockSpec((B, 1), zero2),
            pl.BlockSpec((3, BS, S), lambda i: (0, 0, 0)),
            pl.BlockSpec((3, S, BS), lambda i: (0, 0, 0)),
            pl.BlockSpec((BS, 1), zero2),
            pl.BlockSpec((1, 1), zero2, memory_space=pltpu.SMEM),
            pl.BlockSpec((BS, B), zero2),
            pl.BlockSpec((1, ND), zero2),
            pl.BlockSpec((1, ND), zero2),
            pl.BlockSpec((1, 1), zero2, memory_space=pltpu.SMEM),
        ],
        out_specs=pl.BlockSpec((S, ND), lambda i: (i, 0)),
        out_shape=jax.ShapeDtypeStruct((BS, ND), F32),
    )(feat, gw, gb_r, s, q, bng_c, bnb_c, a1, a2, cb1_rep, cb2_r,
      e_mat, wg1, wg2, bg_r)


def _head_body(feat_ref, e_ref, w1_ref, b1_ref, w2_ref, b2_ref, out_ref):
    pooled = jax.lax.dot_general(e_ref[...], feat_ref[...],
                                 (((0,), (0,)), ((), ()))) * (1.0 / S)
    r = jax.nn.relu(jnp.dot(pooled, w1_ref[...],
                            preferred_element_type=F32) + b1_ref[...])
    out_ref[...] = jnp.dot(r, w2_ref[...],
                           preferred_element_type=F32) + b2_ref[...]


def _head(feat, e_mat, w1, b1_r, w2, b2_r):
    return pl.pallas_call(
        _head_body,
        out_shape=jax.ShapeDtypeStruct((B, L), F32),
    )(feat, e_mat, w1, b1_r, w2, b2_r)


def _banded(w3x3, transpose):
    r = jnp.arange(BS)
    i = (r % S)[:, None]
    ip = jnp.arange(S)[None, :]
    mats = []
    for dwk in range(3):
        acc = jnp.zeros((BS, S), F32)
        for dh in range(3):
            kh = (2 - dh) if transpose else dh
            wcol = jnp.repeat(w3x3[:, kh, dwk], S)[:, None]
            acc = acc + jnp.where(ip - i == dh - 1, wcol, 0.0)
        mats.append(acc.T if transpose else acc)
    return jnp.stack(mats).astype(F32)


def _dense_forward(seq, e1_mask, e2_mask, W_ep, b_ep, blocks,
                   W_gate, b_gate, Wc1, bc1, Wc2, bc2):
    e_mat = (jnp.arange(BS)[:, None] // S == jnp.arange(B)[None, :]).astype(F32)
    feat = _feat(seq, e1_mask, e2_mask, W_ep, b_ep.reshape(1, ENT), e_mat)

    wg1 = W_gate[:ND, 0].reshape(1, ND)
    wg2 = W_gate[ND:, 0].reshape(1, ND)
    bg_r = b_gate.reshape(1, 1)

    for (gw, gb, cw1, cb1, bng, bnb, cw2, cb2) in blocks:
        a1 = _banded(cw1.reshape(CH, 3, 3), transpose=False)
        a2 = _banded(cw2.reshape(CH, 3, 3), transpose=True)
        cb1_rep = jnp.repeat(cb1, S).reshape(BS, 1).astype(F32)
        s, q = _stats(feat, a1, cb1_rep, e_mat)
        feat = _block(feat, gw, gb.reshape(1, ND), s, q,
                      bng.reshape(B, 1), bnb.reshape(B, 1), a1, a2,
                      cb1_rep, cb2.reshape(1, 1), e_mat, wg1, wg2, bg_r)

    return _head(feat, e_mat, Wc1, bc1.reshape(1, ND), Wc2, bc2.reshape(1, L))


def kernel(input_ids, e1_mask, e2_mask, emb, W_ep, b_ep,
           gcn_w0, gcn_b0, cw1_0, cb1_0, bng0, bnb0, cw2_0, cb2_0,
           gcn_w1, gcn_b1, cw1_1, cb1_1, bng1, bnb1, cw2_1, cb2_1,
           W_gate, b_gate, Wc1, bc1, Wc2, bc2):
    ids = input_ids.reshape(BS).astype(jnp.int32)
    seq = _sc_gather(emb, ids)
    blocks = [(gcn_w0, gcn_b0, cw1_0, cb1_0, bng0, bnb0, cw2_0, cb2_0),
              (gcn_w1, gcn_b1, cw1_1, cb1_1, bng1, bnb1, cw2_1, cb2_1)]
    return _dense_forward(seq, e1_mask, e2_mask, W_ep, b_ep, blocks,
                          W_gate, b_gate, Wc1, bc1, Wc2, bc2)
